# Initial kernel scaffold; baseline (speedup 1.0000x reference)
import jax
import jax.numpy as jnp
from jax.experimental import pallas as pl


def kernel(node_species, element_indices, embed_table):
    raise NotImplementedError("write your pallas kernel here")


# speedup vs baseline: 1.2533x; 1.2533x over previous
import functools

import jax
import jax.numpy as jnp
from jax import lax
from jax.experimental import pallas as pl
from jax.experimental.pallas import tpu as pltpu
from jax.experimental.pallas import tpu_sc as plsc

N_NODES = 100000
OUT_DIM = 128
LANES = 16
NUM_WORKERS = 32
BLK = 128
BLKS_PER_W = 25
PER_W = BLK * BLKS_PER_W
N_PAD = NUM_WORKERS * PER_W


def _sc_embed(ns_hbm, elem_hbm, emb_hbm, out_hbm,
              ns_v, elem_v, spec_v, rows0, rows1,
              gsem0, gsem1, ssem0, ssem1):
    wid = lax.axis_index("s") * 2 + lax.axis_index("c")
    base = wid * PER_W

    pltpu.sync_copy(ns_hbm.at[pl.ds(base, PER_W)], ns_v)
    pltpu.sync_copy(elem_hbm, elem_v)

    rows = (rows0, rows1)
    gsems = (gsem0, gsem1)
    ssems = (ssem0, ssem1)
    scat = [None, None]
    for b in range(BLKS_PER_W):
        i = b & 1
        for j in range(BLK // LANES):
            idx = ns_v[pl.ds(b * BLK + j * LANES, LANES)]
            spec_v[b, pl.ds(j * LANES, LANES)] = plsc.load_gather(elem_v, [idx])
        if scat[i] is not None:
            scat[i].wait()
        g = pltpu.async_copy(emb_hbm.at[spec_v.at[b]], rows[i], gsems[i])
        g.wait()
        scat[i] = pltpu.async_copy(
            rows[i], out_hbm.at[pl.ds(base + b * BLK, BLK)], ssems[i])
    scat[0].wait()
    scat[1].wait()


@jax.jit
def _run(ns_pad, elem_pad, emb):
    mesh = plsc.VectorSubcoreMesh(core_axis_name="c", subcore_axis_name="s")
    f = functools.partial(
        pl.kernel,
        mesh=mesh,
        compiler_params=pltpu.CompilerParams(needs_layout_passes=False),
        out_type=jax.ShapeDtypeStruct((N_PAD, OUT_DIM), jnp.float32),
        scratch_types=[
            pltpu.VMEM((PER_W,), jnp.int32),
            pltpu.VMEM((OUT_DIM,), jnp.int32),
            pltpu.VMEM((BLKS_PER_W, BLK), jnp.int32),
            pltpu.VMEM((BLK, OUT_DIM), jnp.float32),
            pltpu.VMEM((BLK, OUT_DIM), jnp.float32),
            pltpu.SemaphoreType.DMA,
            pltpu.SemaphoreType.DMA,
            pltpu.SemaphoreType.DMA,
            pltpu.SemaphoreType.DMA,
        ],
    )(_sc_embed)
    return f(ns_pad, elem_pad, emb)


def kernel(node_species, element_indices, embed_table):
    ns = jnp.asarray(node_species, jnp.int32)
    ns_pad = jnp.pad(ns, (0, N_PAD - N_NODES))
    elem_pad = jnp.pad(jnp.asarray(element_indices, jnp.int32),
                       (0, OUT_DIM - element_indices.shape[0]))
    out = _run(ns_pad, elem_pad, jnp.asarray(embed_table, jnp.float32))
    return out[:N_NODES]


# speedup vs baseline: 10.0772x; 8.0405x over previous
import functools

import jax
import jax.numpy as jnp
from jax import lax
from jax.experimental import pallas as pl
from jax.experimental.pallas import tpu as pltpu
from jax.experimental.pallas import tpu_sc as plsc

N_NODES = 100000
OUT_DIM = 128
LANES = 16
NUM_WORKERS = 32
BLK = 128
BLKS_PER_W = 25
PER_W = BLK * BLKS_PER_W
N_PAD = NUM_WORKERS * PER_W


def _sc_embed(ns_hbm, elem_hbm, emb_hbm, out_hbm,
              ns_v, elem_v, spec_v, table_v, rows0, rows1,
              gsem0, gsem1, ssem0, ssem1):
    wid = lax.axis_index("s") * 2 + lax.axis_index("c")
    base = wid * PER_W

    pltpu.sync_copy(ns_hbm.at[pl.ds(base, PER_W)], ns_v)
    pltpu.sync_copy(elem_hbm, elem_v)
    @pl.when(lax.axis_index("s") == 0)
    def _():
        pltpu.sync_copy(emb_hbm, table_v)
    plsc.subcore_barrier()

    rows = (rows0, rows1)
    gsems = (gsem0, gsem1)
    ssems = (ssem0, ssem1)
    scat = [None, None]
    for b in range(BLKS_PER_W):
        i = b & 1
        for j in range(BLK // LANES):
            idx = ns_v[pl.ds(b * BLK + j * LANES, LANES)]
            spec_v[b, pl.ds(j * LANES, LANES)] = plsc.load_gather(elem_v, [idx])
        if scat[i] is not None:
            scat[i].wait()
        g = pltpu.async_copy(table_v.at[spec_v.at[b]], rows[i], gsems[i])
        g.wait()
        scat[i] = pltpu.async_copy(
            rows[i], out_hbm.at[pl.ds(base + b * BLK, BLK)], ssems[i])
    scat[0].wait()
    scat[1].wait()


@jax.jit
def _run(ns_pad, elem_pad, emb):
    mesh = plsc.VectorSubcoreMesh(core_axis_name="c", subcore_axis_name="s")
    f = functools.partial(
        pl.kernel,
        mesh=mesh,
        compiler_params=pltpu.CompilerParams(needs_layout_passes=False),
        out_type=jax.ShapeDtypeStruct((N_PAD, OUT_DIM), jnp.float32),
        scratch_types=[
            pltpu.VMEM((PER_W,), jnp.int32),
            pltpu.VMEM((OUT_DIM,), jnp.int32),
            pltpu.VMEM((BLKS_PER_W, BLK), jnp.int32),
            pltpu.VMEM_SHARED((10, OUT_DIM), jnp.float32),
            pltpu.VMEM((BLK, OUT_DIM), jnp.float32),
            pltpu.VMEM((BLK, OUT_DIM), jnp.float32),
            pltpu.SemaphoreType.DMA,
            pltpu.SemaphoreType.DMA,
            pltpu.SemaphoreType.DMA,
            pltpu.SemaphoreType.DMA,
        ],
    )(_sc_embed)
    return f(ns_pad, elem_pad, emb)


def kernel(node_species, element_indices, embed_table):
    ns = jnp.asarray(node_species, jnp.int32)
    ns_pad = jnp.pad(ns, (0, N_PAD - N_NODES))
    elem_pad = jnp.pad(jnp.asarray(element_indices, jnp.int32),
                       (0, OUT_DIM - element_indices.shape[0]))
    out = _run(ns_pad, elem_pad, jnp.asarray(embed_table, jnp.float32))
    return out[:N_NODES]


# speedup vs baseline: 17.0108x; 1.6880x over previous
import functools

import jax
import jax.numpy as jnp
from jax import lax
from jax.experimental import pallas as pl
from jax.experimental.pallas import tpu as pltpu
from jax.experimental.pallas import tpu_sc as plsc

N_NODES = 100000
OUT_DIM = 128
LANES = 16
NUM_WORKERS = 32
BLK = 125
BLKS_PER_W = 25
PER_W = BLK * BLKS_PER_W
WIN = PER_W + 11
N_PAD = 100008
GROUPS = [0, 16, 32, 48, 64, 80, 96, BLK - LANES]


def _sc_embed(ns_hbm, elem_hbm, emb_hbm, out_hbm,
              ns_v, elem_v, spec_v, table_s, rows0, rows1,
              gsem0, gsem1, ssem0, ssem1):
    wid = lax.axis_index("s") * 2 + lax.axis_index("c")
    base = wid * PER_W
    start = pl.multiple_of(8 * (base // 8), 8)
    delta = base - start

    pltpu.sync_copy(ns_hbm.at[pl.ds(start, WIN)], ns_v)
    pltpu.sync_copy(elem_hbm, elem_v)

    @pl.when(lax.axis_index("s") == 0)
    def _():
        pltpu.sync_copy(emb_hbm, table_s)
    plsc.subcore_barrier()

    rows = (rows0, rows1)
    gsems = (gsem0, gsem1)
    ssems = (ssem0, ssem1)
    scat = [None, None]
    for b in range(BLKS_PER_W):
        i = b & 1
        for off in GROUPS:
            idx = ns_v[pl.ds(delta + b * BLK + off, LANES)]
            spec_v[b, pl.ds(off, LANES)] = plsc.load_gather(elem_v, [idx])
        if scat[i] is not None:
            scat[i].wait()
        g = pltpu.async_copy(table_s.at[spec_v.at[b]], rows[i], gsems[i])
        g.wait()
        scat[i] = pltpu.async_copy(
            rows[i], out_hbm.at[pl.ds(base + b * BLK, BLK)], ssems[i])
    scat[0].wait()
    scat[1].wait()


@jax.jit
def _run(ns_pad, elem_pad, emb):
    mesh = plsc.VectorSubcoreMesh(core_axis_name="c", subcore_axis_name="s")
    f = functools.partial(
        pl.kernel,
        mesh=mesh,
        compiler_params=pltpu.CompilerParams(
            needs_layout_passes=False, use_tc_tiling_on_sc=False),
        out_type=jax.ShapeDtypeStruct((N_NODES, OUT_DIM), jnp.float32),
        scratch_types=[
            pltpu.VMEM((WIN,), jnp.int32),
            pltpu.VMEM((OUT_DIM,), jnp.int32),
            pltpu.VMEM((BLKS_PER_W, BLK), jnp.int32),
            pltpu.VMEM_SHARED((10, OUT_DIM), jnp.float32),
            pltpu.VMEM((BLK, OUT_DIM), jnp.float32),
            pltpu.VMEM((BLK, OUT_DIM), jnp.float32),
            pltpu.SemaphoreType.DMA,
            pltpu.SemaphoreType.DMA,
            pltpu.SemaphoreType.DMA,
            pltpu.SemaphoreType.DMA,
        ],
    )(_sc_embed)
    return f(ns_pad, elem_pad, emb)


def kernel(node_species, element_indices, embed_table):
    ns = jnp.asarray(node_species, jnp.int32)
    ns_pad = jnp.pad(ns, (0, N_PAD - N_NODES))
    elem_pad = jnp.pad(jnp.asarray(element_indices, jnp.int32),
                       (0, OUT_DIM - element_indices.shape[0]))
    return _run(ns_pad, elem_pad, jnp.asarray(embed_table, jnp.float32))
